# Initial kernel scaffold; baseline (speedup 1.0000x reference)
import jax
import jax.numpy as jnp
from jax.experimental import pallas as pl


def kernel(center, outside, all_vocabs, W_center, W_outside):
    raise NotImplementedError("write your pallas kernel here")


# speedup vs baseline: 70.5067x; 70.5067x over previous
import jax
import jax.numpy as jnp
from jax import lax
from jax.experimental import pallas as pl
from jax.experimental.pallas import tpu as pltpu
from jax.experimental.pallas import tpu_sc as plsc

B = 1024
V = 1000
E = 64
VP = 1024
NW = 32
BPW = B // NW
NFULL = V // 16
NTAIL = V - NFULL * 16


def _gram_exp_body(w_ref, out_ref):
    w = w_ref[...]
    g = lax.dot_general(w, w, (((1,), (1,)), ((), ())),
                        preferred_element_type=jnp.float32,
                        precision=lax.Precision.HIGHEST)
    out_ref[:, :V] = jnp.exp(g)


def _gram_exp(w):
    return pl.pallas_call(
        _gram_exp_body,
        out_shape=jax.ShapeDtypeStruct((V, VP), jnp.float32),
    )(w)


def _sc_body(eg_hbm, center_hbm, av_hbm, outside_hbm,
             lacc_hbm, topv_hbm,
             center_v, eg_v, av_v, outside_v, lacc_v, topv_v, sem, sem2):
    nc = 2
    wid = lax.axis_index("s") * nc + lax.axis_index("c")
    base = wid * BPW

    pltpu.sync_copy(center_hbm.at[pl.ds(base, BPW)], center_v)
    pltpu.sync_copy(outside_hbm.at[pl.ds(base, BPW)], outside_v)
    rows_cp = pltpu.async_copy(eg_hbm.at[center_v], eg_v, sem)
    av_cp = pltpu.async_copy(av_hbm.at[pl.ds(base * V, BPW * V)],
                             av_v.at[pl.ds(0, BPW * V)], sem2)
    rows_cp.wait()
    av_cp.wait()

    lane = lax.iota(jnp.int32, 16)
    tailmask = lane < NTAIL

    def row_body(i, _):
        row_vec = jnp.full((16,), i, dtype=jnp.int32)

        def chunk_body(j, acc):
            col = av_v[pl.ds(i * V + j * 16, 16)]
            return acc + plsc.load_gather(eg_v, [row_vec, col])

        acc = lax.fori_loop(0, NFULL, chunk_body,
                            jnp.zeros((16,), jnp.float32), unroll=2)
        colt = av_v[pl.ds(i * V + NFULL * 16, 16)]
        gt = plsc.load_gather(eg_v, [row_vec, colt], mask=tailmask)
        acc = acc + jnp.where(tailmask, gt, 0.0)
        lacc_v[i, :] = acc
        return 0

    lax.fori_loop(0, BPW, row_body, 0)

    for g in range(BPW // 16):
        rows = g * 16 + lane
        cols = outside_v[pl.ds(g * 16, 16)]
        topv_v[pl.ds(g * 16, 16)] = plsc.load_gather(eg_v, [rows, cols])

    pltpu.sync_copy(lacc_v, lacc_hbm.at[pl.ds(base, BPW)])
    pltpu.sync_copy(topv_v, topv_hbm.at[pl.ds(base, BPW)])


def _sc_gather_sum(eg, center_flat, av_flat, outside_flat):
    mesh = plsc.VectorSubcoreMesh(core_axis_name="c", subcore_axis_name="s",
                                  num_cores=2, num_subcores=16)
    f = pl.kernel(
        _sc_body,
        out_type=(
            jax.ShapeDtypeStruct((B, 16), jnp.float32),
            jax.ShapeDtypeStruct((B,), jnp.float32),
        ),
        mesh=mesh,
        scratch_types=[
            pltpu.VMEM((BPW,), jnp.int32),
            pltpu.VMEM((BPW, VP), jnp.float32),
            pltpu.VMEM((BPW * V + 16,), jnp.int32),
            pltpu.VMEM((BPW,), jnp.int32),
            pltpu.VMEM((BPW, 16), jnp.float32),
            pltpu.VMEM((BPW,), jnp.float32),
            pltpu.SemaphoreType.DMA,
            pltpu.SemaphoreType.DMA,
        ],
        compiler_params=pltpu.CompilerParams(use_tc_tiling_on_sc=False,
                                             needs_layout_passes=False),
    )
    return f(eg, center_flat, av_flat, outside_flat)


def _loss_body(lacc_ref, topv_ref, out_ref):
    lower = jnp.sum(lacc_ref[...], axis=1)
    loss = jnp.mean(jnp.log(lower)) - jnp.mean(jnp.log(topv_ref[...]))
    out_ref[0, 0] = loss


def _final_loss(lacc, topv2d):
    out = pl.pallas_call(
        _loss_body,
        out_shape=jax.ShapeDtypeStruct((1, 1), jnp.float32),
        out_specs=pl.BlockSpec(memory_space=pltpu.SMEM),
    )(lacc, topv2d)
    return out[0, 0]


@jax.jit
def kernel(center, outside, all_vocabs, W_center, W_outside):
    del W_outside
    eg = _gram_exp(W_center)

    center_flat = center.reshape(B).astype(jnp.int32)
    outside_flat = outside.reshape(B).astype(jnp.int32)
    av_flat = all_vocabs.astype(jnp.int32).reshape(B * V)

    lacc, topv = _sc_gather_sum(eg, center_flat, av_flat, outside_flat)
    return _final_loss(lacc, topv.reshape(8, 128))
